# baseline (speedup 1.0000x reference)
import functools

import jax
import jax.numpy as jnp
from jax import lax
from jax.experimental import pallas as pl
from jax.experimental.pallas import tpu as pltpu
from jax.experimental.pallas import tpu_sc as plsc

E = 8
TOPK = 2
D = 1024
F = 4096
T = 2048

BM = 1024
P = T * TOPK + E * BM
NT = P // BM
FB = 2048
NF = F // FB
NW = 32
CH = T // NW


def _gate_body(x_ref, gw_ref, pos_ref, w_ref, gid_ref):
    x = x_ref[...]
    gw = gw_ref[...]
    logits = lax.dot_general(x, gw, (((1,), (1,)), ((), ())),
                             preferred_element_type=jnp.float32)
    m = jnp.max(logits, axis=1, keepdims=True)
    ex = jnp.exp(logits - m)
    probs = ex / jnp.sum(ex, axis=1, keepdims=True)

    eio = lax.broadcasted_iota(jnp.int32, (T, E), 1)
    m1 = jnp.max(probs, axis=1, keepdims=True)
    i1 = jnp.min(jnp.where(probs == m1, eio, E), axis=1, keepdims=True)
    probs2 = jnp.where(eio == i1, -jnp.inf, probs)
    m2 = jnp.max(probs2, axis=1, keepdims=True)
    i2 = jnp.min(jnp.where(probs2 == m2, eio, E), axis=1, keepdims=True)
    s = m1 + m2
    w_ref[...] = jnp.concatenate([m1 / s, m2 / s], axis=1)

    e2 = jnp.concatenate([i1, i2], axis=1)
    oh = (e2[:, :, None]
          == lax.broadcasted_iota(jnp.int32, (T, TOPK, E), 2)).astype(jnp.int32)
    ohf = oh.reshape(T * TOPK, E)
    c = ohf
    sh = 1
    while sh < T * TOPK:
        z = jnp.zeros((sh, E), jnp.int32)
        c = c + jnp.concatenate([z, c[: T * TOPK - sh, :]], axis=0)
        sh *= 2
    rank = (c - ohf).reshape(T, TOPK, E)
    cnt = c[T * TOPK - 1 :, :]
    cntp = ((cnt + (BM - 1)) // BM) * BM
    ei = lax.broadcasted_iota(jnp.int32, (E, E), 0)
    ej = lax.broadcasted_iota(jnp.int32, (E, E), 1)
    ut = (ei < ej).astype(jnp.float32)
    off = lax.dot_general(cntp.astype(jnp.float32), ut,
                          (((1,), (0,)), ((), ())),
                          precision=lax.Precision.HIGHEST,
                          preferred_element_type=jnp.float32)
    offsel = jnp.sum(oh.astype(jnp.float32) * off[None, :, :], axis=2)
    ranksel = jnp.sum(oh * rank, axis=2)
    pos_ref[...] = offsel.astype(jnp.int32) + ranksel

    starts = (BM * lax.broadcasted_iota(jnp.int32, (NT, E), 0)).astype(jnp.float32)
    offb = jnp.broadcast_to(off, (NT, E))
    gid_ref[...] = (jnp.sum((offb <= starts).astype(jnp.int32), axis=1,
                            keepdims=True) - 1)


def _gate_call(x, gate_w):
    return pl.pallas_call(
        _gate_body,
        out_shape=(
            jax.ShapeDtypeStruct((T, TOPK), jnp.int32),
            jax.ShapeDtypeStruct((T, TOPK), jnp.float32),
            jax.ShapeDtypeStruct((NT, 1), jnp.int32),
        ),
    )(x, gate_w)


@functools.cache
def _sc_mesh():
    return plsc.VectorSubcoreMesh(core_axis_name="c", subcore_axis_name="s")


@functools.cache
def _sc_scatter_build():
    @functools.partial(
        pl.kernel,
        mesh=_sc_mesh(),
        out_type=jax.ShapeDtypeStruct((P, D), jnp.float32),
        scratch_types=[
            pltpu.VMEM((CH,), jnp.int32),
            pltpu.VMEM((CH,), jnp.int32),
            pltpu.VMEM((CH, D), jnp.float32),
            pltpu.SemaphoreType.DMA,
            pltpu.SemaphoreType.DMA,
        ],
    )
    def _sc_scatter(x_hbm, pos0_hbm, pos1_hbm, xs_hbm, idx0_v, idx1_v, rows_v,
                    sem0, sem1):
        wid = lax.axis_index("s") * 2 + lax.axis_index("c")
        base = wid * CH
        pltpu.sync_copy(pos0_hbm.at[wid], idx0_v)
        pltpu.sync_copy(pos1_hbm.at[wid], idx1_v)
        pltpu.sync_copy(x_hbm.at[pl.ds(base, CH)], rows_v)
        c0 = pltpu.async_copy(rows_v, xs_hbm.at[idx0_v], sem0)
        c1 = pltpu.async_copy(rows_v, xs_hbm.at[idx1_v], sem1)
        c0.wait()
        c1.wait()

    return _sc_scatter


def _ffn_body(gid_ref, xs_ref, w1_ref, b1_ref, w2_ref, b2_ref, out_ref):
    j = pl.program_id(1)
    xs = xs_ref[...]
    h = lax.dot_general(xs, w1_ref[0], (((1,), (1,)), ((), ())),
                        precision=lax.Precision.DEFAULT,
                        preferred_element_type=jnp.float32)
    h = h + b1_ref[0][0][None, :]
    h = jax.nn.gelu(h)
    p = lax.dot_general(h, w2_ref[0], (((1,), (0,)), ((), ())),
                        precision=lax.Precision.DEFAULT,
                        preferred_element_type=jnp.float32)

    @pl.when(j == 0)
    def _():
        out_ref[...] = p + b2_ref[0][0][None, :]

    @pl.when(j > 0)
    def _():
        out_ref[...] += p


def _ffn_call(gid, xs, fc1_w, fc1_b, fc2_w, fc2_b):
    b1r = fc1_b.reshape(E * NF, 1, FB)
    b2r = fc2_b.reshape(E, 1, D)
    grid_spec = pltpu.PrefetchScalarGridSpec(
        num_scalar_prefetch=1,
        grid=(NT, NF),
        in_specs=[
            pl.BlockSpec((BM, D), lambda i, j, g: (i, 0)),
            pl.BlockSpec((1, FB, D), lambda i, j, g: (g[i], j, 0)),
            pl.BlockSpec((1, 1, FB), lambda i, j, g: (g[i] * NF + j, 0, 0)),
            pl.BlockSpec((1, FB, D), lambda i, j, g: (g[i], j, 0)),
            pl.BlockSpec((1, 1, D), lambda i, j, g: (g[i], 0, 0)),
        ],
        out_specs=pl.BlockSpec((BM, D), lambda i, j, g: (i, 0)),
    )
    return pl.pallas_call(
        _ffn_body,
        grid_spec=grid_spec,
        out_shape=jax.ShapeDtypeStruct((P, D), jnp.float32),
    )(gid, xs, fc1_w, b1r, fc2_w, b2r)


@functools.cache
def _sc_gather_build():
    @functools.partial(
        pl.kernel,
        mesh=_sc_mesh(),
        out_type=(
            jax.ShapeDtypeStruct((T, D), jnp.float32),
            jax.ShapeDtypeStruct((T, D), jnp.float32),
        ),
        scratch_types=[
            pltpu.VMEM((CH,), jnp.int32),
            pltpu.VMEM((CH,), jnp.int32),
            pltpu.VMEM((CH, D), jnp.float32),
            pltpu.SemaphoreType.DMA,
        ],
    )
    def _sc_gather(ys_hbm, pos0_hbm, pos1_hbm, y0_hbm, y1_hbm, idx0_v, idx1_v,
                   buf_v, sem):
        wid = lax.axis_index("s") * 2 + lax.axis_index("c")
        base = wid * CH
        pltpu.sync_copy(pos0_hbm.at[wid], idx0_v)
        pltpu.sync_copy(pos1_hbm.at[wid], idx1_v)
        pltpu.async_copy(ys_hbm.at[idx0_v], buf_v, sem).wait()
        pltpu.sync_copy(buf_v, y0_hbm.at[pl.ds(base, CH)])
        pltpu.async_copy(ys_hbm.at[idx1_v], buf_v, sem).wait()
        pltpu.sync_copy(buf_v, y1_hbm.at[pl.ds(base, CH)])

    return _sc_gather


def _combine_body(w_ref, y0_ref, y1_ref, out_ref):
    w = w_ref[...]
    out_ref[...] = w[:, 0:1] * y0_ref[...] + w[:, 1:2] * y1_ref[...]


def _combine_call(w01, y0, y1):
    return pl.pallas_call(
        _combine_body,
        grid=(T // BM,),
        in_specs=[
            pl.BlockSpec((BM, TOPK), lambda i: (i, 0)),
            pl.BlockSpec((BM, D), lambda i: (i, 0)),
            pl.BlockSpec((BM, D), lambda i: (i, 0)),
        ],
        out_specs=pl.BlockSpec((BM, D), lambda i: (i, 0)),
        out_shape=jax.ShapeDtypeStruct((T, D), jnp.float32),
    )(w01, y0, y1)


def kernel(x, gate_w, fc1_w, fc1_b, fc2_w, fc2_b):
    pos, w01, gid2 = _gate_call(x, gate_w)
    gid = gid2.reshape(NT)
    pos0 = pos[:, 0].reshape(NW, CH)
    pos1 = pos[:, 1].reshape(NW, CH)
    xs = _sc_scatter_build()(x, pos0, pos1)
    ys = _ffn_call(gid, xs, fc1_w, fc1_b, fc2_w, fc2_b)
    y0, y1 = _sc_gather_build()(ys, pos0, pos1)
    return _combine_call(w01, y0, y1)


# speedup vs baseline: 1.1413x; 1.1413x over previous
import functools

import jax
import jax.numpy as jnp
from jax import lax
from jax.experimental import pallas as pl
from jax.experimental.pallas import tpu as pltpu
from jax.experimental.pallas import tpu_sc as plsc

E = 8
TOPK = 2
D = 1024
F = 4096
T = 2048

BM = 512
P = T * TOPK + E * BM
NT = P // BM
FB = 2048
NF = F // FB
NW = 32
CH = T // NW


def _gate_body(x_ref, gw_ref, pos_ref, w_ref, gid_ref):
    x = x_ref[...]
    gw = gw_ref[...]
    logits = lax.dot_general(x, gw, (((1,), (1,)), ((), ())),
                             preferred_element_type=jnp.float32)
    m = jnp.max(logits, axis=1, keepdims=True)
    ex = jnp.exp(logits - m)
    probs = ex / jnp.sum(ex, axis=1, keepdims=True)

    eio = lax.broadcasted_iota(jnp.int32, (T, E), 1)
    m1 = jnp.max(probs, axis=1, keepdims=True)
    i1 = jnp.min(jnp.where(probs == m1, eio, E), axis=1, keepdims=True)
    probs2 = jnp.where(eio == i1, -jnp.inf, probs)
    m2 = jnp.max(probs2, axis=1, keepdims=True)
    i2 = jnp.min(jnp.where(probs2 == m2, eio, E), axis=1, keepdims=True)
    s = m1 + m2
    w_ref[...] = jnp.concatenate([m1 / s, m2 / s], axis=1)

    e2 = jnp.concatenate([i1, i2], axis=1)
    oh = (e2[:, :, None]
          == lax.broadcasted_iota(jnp.int32, (T, TOPK, E), 2)).astype(jnp.int32)
    ohf = oh.reshape(T * TOPK, E)
    c = ohf
    sh = 1
    while sh < T * TOPK:
        z = jnp.zeros((sh, E), jnp.int32)
        c = c + jnp.concatenate([z, c[: T * TOPK - sh, :]], axis=0)
        sh *= 2
    rank = (c - ohf).reshape(T, TOPK, E)
    cnt = c[T * TOPK - 1 :, :]
    cntp = ((cnt + (BM - 1)) // BM) * BM
    ei = lax.broadcasted_iota(jnp.int32, (E, E), 0)
    ej = lax.broadcasted_iota(jnp.int32, (E, E), 1)
    ut = (ei < ej).astype(jnp.float32)
    off = lax.dot_general(cntp.astype(jnp.float32), ut,
                          (((1,), (0,)), ((), ())),
                          precision=lax.Precision.HIGHEST,
                          preferred_element_type=jnp.float32)
    offsel = jnp.sum(oh.astype(jnp.float32) * off[None, :, :], axis=2)
    ranksel = jnp.sum(oh * rank, axis=2)
    pos_ref[...] = offsel.astype(jnp.int32) + ranksel

    starts = (BM * lax.broadcasted_iota(jnp.int32, (NT, E), 0)).astype(jnp.float32)
    offb = jnp.broadcast_to(off, (NT, E))
    gid_ref[...] = (jnp.sum((offb <= starts).astype(jnp.int32), axis=1,
                            keepdims=True) - 1)


def _gate_call(x, gate_w):
    return pl.pallas_call(
        _gate_body,
        out_shape=(
            jax.ShapeDtypeStruct((T, TOPK), jnp.int32),
            jax.ShapeDtypeStruct((T, TOPK), jnp.float32),
            jax.ShapeDtypeStruct((NT, 1), jnp.int32),
        ),
    )(x, gate_w)


@functools.cache
def _sc_mesh():
    return plsc.VectorSubcoreMesh(core_axis_name="c", subcore_axis_name="s")


@functools.cache
def _sc_scatter_build():
    @functools.partial(
        pl.kernel,
        mesh=_sc_mesh(),
        out_type=jax.ShapeDtypeStruct((P, D), jnp.float32),
        scratch_types=[
            pltpu.VMEM((CH,), jnp.int32),
            pltpu.VMEM((CH,), jnp.int32),
            pltpu.VMEM((CH, D), jnp.float32),
            pltpu.SemaphoreType.DMA,
            pltpu.SemaphoreType.DMA,
        ],
    )
    def _sc_scatter(x_hbm, pos0_hbm, pos1_hbm, xs_hbm, idx0_v, idx1_v, rows_v,
                    sem0, sem1):
        wid = lax.axis_index("s") * 2 + lax.axis_index("c")
        base = wid * CH
        pltpu.sync_copy(pos0_hbm.at[wid], idx0_v)
        pltpu.sync_copy(pos1_hbm.at[wid], idx1_v)
        pltpu.sync_copy(x_hbm.at[pl.ds(base, CH)], rows_v)
        c0 = pltpu.async_copy(rows_v, xs_hbm.at[idx0_v], sem0)
        c1 = pltpu.async_copy(rows_v, xs_hbm.at[idx1_v], sem1)
        c0.wait()
        c1.wait()

    return _sc_scatter


def _ffn_body(gid_ref, xs_ref, w1_ref, b1_ref, w2_ref, b2_ref, out_ref):
    j = pl.program_id(1)
    xs = xs_ref[...]
    h = lax.dot_general(xs, w1_ref[0], (((1,), (1,)), ((), ())),
                        precision=lax.Precision.DEFAULT,
                        preferred_element_type=jnp.float32)
    h = h + b1_ref[0][0][None, :]
    h = jax.nn.gelu(h)
    p = lax.dot_general(h.astype(jnp.bfloat16), w2_ref[0],
                        (((1,), (0,)), ((), ())),
                        precision=lax.Precision.DEFAULT,
                        preferred_element_type=jnp.float32)

    @pl.when(j == 0)
    def _():
        out_ref[...] = p + b2_ref[0][0][None, :]

    @pl.when(j > 0)
    def _():
        out_ref[...] += p


def _ffn_call(gid, xs, fc1_w, fc1_b, fc2_w, fc2_b):
    b1r = fc1_b.reshape(E * NF, 1, FB)
    b2r = fc2_b.reshape(E, 1, D)
    grid_spec = pltpu.PrefetchScalarGridSpec(
        num_scalar_prefetch=1,
        grid=(NT, NF),
        in_specs=[
            pl.BlockSpec((BM, D), lambda i, j, g: (i, 0)),
            pl.BlockSpec((1, FB, D), lambda i, j, g: (g[i], j, 0)),
            pl.BlockSpec((1, 1, FB), lambda i, j, g: (g[i] * NF + j, 0, 0)),
            pl.BlockSpec((1, FB, D), lambda i, j, g: (g[i], j, 0)),
            pl.BlockSpec((1, 1, D), lambda i, j, g: (g[i], 0, 0)),
        ],
        out_specs=pl.BlockSpec((BM, D), lambda i, j, g: (i, 0)),
    )
    return pl.pallas_call(
        _ffn_body,
        grid_spec=grid_spec,
        out_shape=jax.ShapeDtypeStruct((P, D), jnp.float32),
    )(gid, xs, fc1_w, b1r, fc2_w, b2r)


@functools.cache
def _sc_gather_build():
    @functools.partial(
        pl.kernel,
        mesh=_sc_mesh(),
        out_type=(
            jax.ShapeDtypeStruct((T, D), jnp.float32),
            jax.ShapeDtypeStruct((T, D), jnp.float32),
        ),
        scratch_types=[
            pltpu.VMEM((CH,), jnp.int32),
            pltpu.VMEM((CH,), jnp.int32),
            pltpu.VMEM((CH, D), jnp.float32),
            pltpu.SemaphoreType.DMA,
        ],
    )
    def _sc_gather(ys_hbm, pos0_hbm, pos1_hbm, y0_hbm, y1_hbm, idx0_v, idx1_v,
                   buf_v, sem):
        wid = lax.axis_index("s") * 2 + lax.axis_index("c")
        base = wid * CH
        pltpu.sync_copy(pos0_hbm.at[wid], idx0_v)
        pltpu.sync_copy(pos1_hbm.at[wid], idx1_v)
        pltpu.async_copy(ys_hbm.at[idx0_v], buf_v, sem).wait()
        pltpu.sync_copy(buf_v, y0_hbm.at[pl.ds(base, CH)])
        pltpu.async_copy(ys_hbm.at[idx1_v], buf_v, sem).wait()
        pltpu.sync_copy(buf_v, y1_hbm.at[pl.ds(base, CH)])

    return _sc_gather


def _combine_body(w_ref, y0_ref, y1_ref, out_ref):
    w = w_ref[...]
    out_ref[...] = w[:, 0:1] * y0_ref[...] + w[:, 1:2] * y1_ref[...]


def _combine_call(w01, y0, y1):
    return pl.pallas_call(
        _combine_body,
        grid=(T // BM,),
        in_specs=[
            pl.BlockSpec((BM, TOPK), lambda i: (i, 0)),
            pl.BlockSpec((BM, D), lambda i: (i, 0)),
            pl.BlockSpec((BM, D), lambda i: (i, 0)),
        ],
        out_specs=pl.BlockSpec((BM, D), lambda i: (i, 0)),
        out_shape=jax.ShapeDtypeStruct((T, D), jnp.float32),
    )(w01, y0, y1)


def kernel(x, gate_w, fc1_w, fc1_b, fc2_w, fc2_b):
    pos, w01, gid2 = _gate_call(x, gate_w)
    gid = gid2.reshape(NT)
    pos0 = pos[:, 0].reshape(NW, CH)
    pos1 = pos[:, 1].reshape(NW, CH)
    xs = _sc_scatter_build()(x, pos0, pos1)
    ys = _ffn_call(gid, xs, fc1_w, fc1_b, fc2_w, fc2_b)
    y0, y1 = _sc_gather_build()(ys, pos0, pos1)
    return _combine_call(w01, y0, y1)


# speedup vs baseline: 1.1450x; 1.0032x over previous
import functools

import jax
import jax.numpy as jnp
from jax import lax
from jax.experimental import pallas as pl
from jax.experimental.pallas import tpu as pltpu
from jax.experimental.pallas import tpu_sc as plsc

E = 8
TOPK = 2
D = 1024
F = 4096
T = 2048

BM = 512
P = T * TOPK + E * BM
NT = P // BM
FB = 2048
NF = F // FB
NW = 32
CH = T // NW


def _gate_body(x_ref, gw_ref, pos_ref, w_ref, gid_ref):
    x = x_ref[...]
    gw = gw_ref[...]
    logits = lax.dot_general(x, gw, (((1,), (1,)), ((), ())),
                             preferred_element_type=jnp.float32)
    m = jnp.max(logits, axis=1, keepdims=True)
    ex = jnp.exp(logits - m)
    probs = ex / jnp.sum(ex, axis=1, keepdims=True)

    eio = lax.broadcasted_iota(jnp.int32, (T, E), 1)
    m1 = jnp.max(probs, axis=1, keepdims=True)
    i1 = jnp.min(jnp.where(probs == m1, eio, E), axis=1, keepdims=True)
    probs2 = jnp.where(eio == i1, -jnp.inf, probs)
    m2 = jnp.max(probs2, axis=1, keepdims=True)
    i2 = jnp.min(jnp.where(probs2 == m2, eio, E), axis=1, keepdims=True)
    s = m1 + m2
    w_ref[...] = jnp.concatenate([m1 / s, m2 / s], axis=1)

    e2 = jnp.concatenate([i1, i2], axis=1)
    oh = (e2[:, :, None]
          == lax.broadcasted_iota(jnp.int32, (T, TOPK, E), 2)).astype(jnp.int32)
    ohf = oh.reshape(T * TOPK, E)
    c = ohf
    sh = 1
    while sh < T * TOPK:
        z = jnp.zeros((sh, E), jnp.int32)
        c = c + jnp.concatenate([z, c[: T * TOPK - sh, :]], axis=0)
        sh *= 2
    rank = (c - ohf).reshape(T, TOPK, E)
    cnt = c[T * TOPK - 1 :, :]
    cntp = ((cnt + (BM - 1)) // BM) * BM
    ei = lax.broadcasted_iota(jnp.int32, (E, E), 0)
    ej = lax.broadcasted_iota(jnp.int32, (E, E), 1)
    ut = (ei < ej).astype(jnp.float32)
    off = lax.dot_general(cntp.astype(jnp.float32), ut,
                          (((1,), (0,)), ((), ())),
                          precision=lax.Precision.HIGHEST,
                          preferred_element_type=jnp.float32)
    offsel = jnp.sum(oh.astype(jnp.float32) * off[None, :, :], axis=2)
    ranksel = jnp.sum(oh * rank, axis=2)
    pos_ref[...] = offsel.astype(jnp.int32) + ranksel

    starts = (BM * lax.broadcasted_iota(jnp.int32, (NT, E), 0)).astype(jnp.float32)
    offb = jnp.broadcast_to(off, (NT, E))
    gid_ref[...] = (jnp.sum((offb <= starts).astype(jnp.int32), axis=1,
                            keepdims=True) - 1)


def _gate_call(x, gate_w):
    return pl.pallas_call(
        _gate_body,
        out_shape=(
            jax.ShapeDtypeStruct((T, TOPK), jnp.int32),
            jax.ShapeDtypeStruct((T, TOPK), jnp.float32),
            jax.ShapeDtypeStruct((NT, 1), jnp.int32),
        ),
    )(x, gate_w)


@functools.cache
def _sc_mesh():
    return plsc.VectorSubcoreMesh(core_axis_name="c", subcore_axis_name="s")


@functools.cache
def _sc_scatter_build():
    @functools.partial(
        pl.kernel,
        mesh=_sc_mesh(),
        out_type=jax.ShapeDtypeStruct((P, D), jnp.float32),
        scratch_types=[
            pltpu.VMEM((CH,), jnp.int32),
            pltpu.VMEM((CH,), jnp.int32),
            pltpu.VMEM((CH, D), jnp.float32),
            pltpu.SemaphoreType.DMA,
            pltpu.SemaphoreType.DMA,
        ],
    )
    def _sc_scatter(x_hbm, pos0_hbm, pos1_hbm, xs_hbm, idx0_v, idx1_v, rows_v,
                    sem0, sem1):
        wid = lax.axis_index("s") * 2 + lax.axis_index("c")
        base = wid * CH
        pltpu.sync_copy(pos0_hbm.at[wid], idx0_v)
        pltpu.sync_copy(pos1_hbm.at[wid], idx1_v)
        pltpu.sync_copy(x_hbm.at[pl.ds(base, CH)], rows_v)
        c0 = pltpu.async_copy(rows_v, xs_hbm.at[idx0_v], sem0)
        c1 = pltpu.async_copy(rows_v, xs_hbm.at[idx1_v], sem1)
        c0.wait()
        c1.wait()

    return _sc_scatter


def _ffn_body(gid_ref, xs_ref, w1_ref, b1_ref, w2_ref, b2_ref, out_ref):
    j = pl.program_id(1)
    xs = xs_ref[...].astype(jnp.bfloat16)
    h = lax.dot_general(xs, w1_ref[0], (((1,), (1,)), ((), ())),
                        precision=lax.Precision.DEFAULT,
                        preferred_element_type=jnp.float32)
    h = h + b1_ref[0][0][None, :]
    h = jax.nn.gelu(h)
    p = lax.dot_general(h.astype(jnp.bfloat16), w2_ref[0],
                        (((1,), (0,)), ((), ())),
                        precision=lax.Precision.DEFAULT,
                        preferred_element_type=jnp.float32)

    @pl.when(j == 0)
    def _():
        out_ref[...] = p + b2_ref[0][0][None, :]

    @pl.when(j > 0)
    def _():
        out_ref[...] += p


def _ffn_call(gid, xs, fc1_w, fc1_b, fc2_w, fc2_b):
    b1r = fc1_b.reshape(E * NF, 1, FB)
    b2r = fc2_b.reshape(E, 1, D)
    grid_spec = pltpu.PrefetchScalarGridSpec(
        num_scalar_prefetch=1,
        grid=(NT, NF),
        in_specs=[
            pl.BlockSpec((BM, D), lambda i, j, g: (i, 0)),
            pl.BlockSpec((1, FB, D), lambda i, j, g: (g[i], j, 0)),
            pl.BlockSpec((1, 1, FB), lambda i, j, g: (g[i] * NF + j, 0, 0)),
            pl.BlockSpec((1, FB, D), lambda i, j, g: (g[i], j, 0)),
            pl.BlockSpec((1, 1, D), lambda i, j, g: (g[i], 0, 0)),
        ],
        out_specs=pl.BlockSpec((BM, D), lambda i, j, g: (i, 0)),
    )
    return pl.pallas_call(
        _ffn_body,
        grid_spec=grid_spec,
        out_shape=jax.ShapeDtypeStruct((P, D), jnp.float32),
    )(gid, xs, fc1_w, b1r, fc2_w, b2r)


@functools.cache
def _sc_gather_build():
    @functools.partial(
        pl.kernel,
        mesh=_sc_mesh(),
        out_type=(
            jax.ShapeDtypeStruct((T, D), jnp.float32),
            jax.ShapeDtypeStruct((T, D), jnp.float32),
        ),
        scratch_types=[
            pltpu.VMEM((CH,), jnp.int32),
            pltpu.VMEM((CH,), jnp.int32),
            pltpu.VMEM((CH, D), jnp.float32),
            pltpu.SemaphoreType.DMA,
        ],
    )
    def _sc_gather(ys_hbm, pos0_hbm, pos1_hbm, y0_hbm, y1_hbm, idx0_v, idx1_v,
                   buf_v, sem):
        wid = lax.axis_index("s") * 2 + lax.axis_index("c")
        base = wid * CH
        pltpu.sync_copy(pos0_hbm.at[wid], idx0_v)
        pltpu.sync_copy(pos1_hbm.at[wid], idx1_v)
        pltpu.async_copy(ys_hbm.at[idx0_v], buf_v, sem).wait()
        pltpu.sync_copy(buf_v, y0_hbm.at[pl.ds(base, CH)])
        pltpu.async_copy(ys_hbm.at[idx1_v], buf_v, sem).wait()
        pltpu.sync_copy(buf_v, y1_hbm.at[pl.ds(base, CH)])

    return _sc_gather


def _combine_body(w_ref, y0_ref, y1_ref, out_ref):
    w = w_ref[...]
    out_ref[...] = w[:, 0:1] * y0_ref[...] + w[:, 1:2] * y1_ref[...]


def _combine_call(w01, y0, y1):
    return pl.pallas_call(
        _combine_body,
        grid=(T // BM,),
        in_specs=[
            pl.BlockSpec((BM, TOPK), lambda i: (i, 0)),
            pl.BlockSpec((BM, D), lambda i: (i, 0)),
            pl.BlockSpec((BM, D), lambda i: (i, 0)),
        ],
        out_specs=pl.BlockSpec((BM, D), lambda i: (i, 0)),
        out_shape=jax.ShapeDtypeStruct((T, D), jnp.float32),
    )(w01, y0, y1)


def kernel(x, gate_w, fc1_w, fc1_b, fc2_w, fc2_b):
    pos, w01, gid2 = _gate_call(x, gate_w)
    gid = gid2.reshape(NT)
    pos0 = pos[:, 0].reshape(NW, CH)
    pos1 = pos[:, 1].reshape(NW, CH)
    xs = _sc_scatter_build()(x, pos0, pos1)
    ys = _ffn_call(gid, xs, fc1_w, fc1_b, fc2_w, fc2_b)
    y0, y1 = _sc_gather_build()(ys, pos0, pos1)
    return _combine_call(w01, y0, y1)
